# Initial kernel scaffold; baseline (speedup 1.0000x reference)
import jax
import jax.numpy as jnp
from jax.experimental import pallas as pl


def kernel(x_user, x_job, edge_index_u2j, edge_index_j2u, edge_label_index, W1l_u2j, W1r_u2j, b1_u2j, W1l_j2u, W1r_j2u, b1_j2u, W2l_u2j, W2r_u2j, b2_u2j, W2l_j2u, W2r_j2u, b2_j2u):
    raise NotImplementedError("write your pallas kernel here")


# speedup vs baseline: 4.0794x; 4.0794x over previous
import functools

import jax
import jax.numpy as jnp
from jax import lax
from jax.experimental import pallas as pl
from jax.experimental.pallas import tpu as pltpu
from jax.experimental.pallas import tpu_sc as plsc

N = 5000
NP = 5120
D = 128
E = 320000
L = 65536
NC = 2
NS = 16
RPT = NP // NS
EPT = E // NS
CH = 80
ITERS = EPT // CH
CW = 16
LPW = L // (NC * NS)
CHL = 128
LITERS = LPW // CHL

_f32 = jnp.float32
_i32 = jnp.int32


def _mesh():
    return plsc.VectorSubcoreMesh(core_axis_name="c", subcore_axis_name="s")


def _make_agg(with_cnt):
    out_type = [jax.ShapeDtypeStruct((NP, D), _f32),
                jax.ShapeDtypeStruct((NP, D), _f32)]
    scratch = [
        pltpu.VMEM_SHARED((NP, D), _f32),
        pltpu.VMEM((CH,), _i32),
        pltpu.VMEM((CH,), _i32),
        pltpu.VMEM((CH, D), _f32),
        pltpu.SemaphoreType.DMA,
    ]
    if with_cnt:
        out_type += [jax.ShapeDtypeStruct((NP, D), _f32),
                     jax.ShapeDtypeStruct((NP, D), _f32)]
        scratch += [
            pltpu.VMEM_SHARED((NP, D), _f32),
            pltpu.VMEM((CH, D), _f32),
        ]

    @functools.partial(pl.kernel, mesh=_mesh(), out_type=out_type,
                       scratch_types=scratch)
    def agg(*refs):
        if with_cnt:
            (xu, xj, su, du, sj, dj, znd, ones,
             out_j, out_u, cnt_j, cnt_u,
             acc, sbuf, dbuf, rbuf, sem,
             cacc, obuf) = refs
        else:
            (xu, xj, su, du, sj, dj, znd,
             out_j, out_u,
             acc, sbuf, dbuf, rbuf, sem) = refs
        cid = lax.axis_index("c")
        sid = lax.axis_index("s")
        r0 = sid * RPT

        def run(x_hbm, src_hbm, dst_hbm, out_hbm, cnt_hbm):
            pltpu.sync_copy(znd.at[pl.ds(r0, RPT)], acc.at[pl.ds(r0, RPT)])
            if with_cnt:
                pltpu.sync_copy(znd.at[pl.ds(r0, RPT)],
                                cacc.at[pl.ds(r0, RPT)])
                pltpu.sync_copy(ones, obuf)
            plsc.subcore_barrier()

            ebase = sid * EPT

            def step(i, carry):
                b = ebase + i * CH
                pltpu.sync_copy(src_hbm.at[pl.ds(b, CH)], sbuf)
                pltpu.sync_copy(dst_hbm.at[pl.ds(b, CH)], dbuf)
                pltpu.async_copy(x_hbm.at[sbuf], rbuf, sem).wait()
                pltpu.sync_copy(rbuf, acc.at[dbuf], add=True)
                if with_cnt:
                    pltpu.sync_copy(obuf, cacc.at[dbuf], add=True)
                return carry

            lax.fori_loop(0, ITERS, step, 0)
            plsc.subcore_barrier()

            pltpu.sync_copy(acc.at[pl.ds(r0, RPT)], out_hbm.at[pl.ds(r0, RPT)])
            if with_cnt:
                pltpu.sync_copy(cacc.at[pl.ds(r0, RPT)],
                                cnt_hbm.at[pl.ds(r0, RPT)])

        @pl.when(cid == 0)
        def _():
            run(xu, su, du, out_j, cnt_j if with_cnt else None)

        @pl.when(cid == 1)
        def _():
            run(xj, sj, dj, out_u, cnt_u if with_cnt else None)

    return agg


@functools.lru_cache(maxsize=None)
def _get_agg(with_cnt):
    return _make_agg(with_cnt)


@functools.lru_cache(maxsize=None)
def _get_gather():
    @functools.partial(
        pl.kernel, mesh=_mesh(),
        out_type=[jax.ShapeDtypeStruct((L, D), _f32),
                  jax.ShapeDtypeStruct((L, D), _f32)],
        scratch_types=[
            pltpu.VMEM((CHL,), _i32),
            pltpu.VMEM((CHL, D), _f32),
            pltpu.SemaphoreType.DMA,
        ])
    def _gather_pairs(zu, zj, la, lb, out_a, out_b, ibuf, rbuf, sem):
        cid = lax.axis_index("c")
        sid = lax.axis_index("s")
        base = (cid * NS + sid) * LPW

        def step(i, carry):
            b = base + i * CHL
            pltpu.sync_copy(la.at[pl.ds(b, CHL)], ibuf)
            pltpu.async_copy(zu.at[ibuf], rbuf, sem).wait()
            pltpu.sync_copy(rbuf, out_a.at[pl.ds(b, CHL)])
            pltpu.sync_copy(lb.at[pl.ds(b, CHL)], ibuf)
            pltpu.async_copy(zj.at[ibuf], rbuf, sem).wait()
            pltpu.sync_copy(rbuf, out_b.at[pl.ds(b, CHL)])
            return carry

        lax.fori_loop(0, LITERS, step, 0)

    return _gather_pairs


def _mm_t(x, w):
    return lax.dot_general(x, w, (((1,), (1,)), ((), ())),
                           preferred_element_type=_f32)


def _sage_body(summ_j, cnt_j, x_j, w_l_j, b_j, w_r_j,
               summ_u, cnt_u, x_u, w_l_u, b_u, w_r_u,
               out_j, out_u, *, relu, normalize):
    def one(summ, cnt, x, wl, b, wr, out):
        mean = summ[...] / jnp.clip(cnt[...][:, 0:1], 1.0, None)
        h = _mm_t(mean, wl[...]) + b[...] + _mm_t(x[...], wr[...])
        if relu:
            h = jnp.maximum(h, 0.0)
        if normalize:
            nrm = jnp.sqrt(jnp.sum(h * h, axis=1, keepdims=True))
            h = h / jnp.clip(nrm, 1e-8, None)
        out[...] = h

    one(summ_j, cnt_j, x_j, w_l_j, b_j, w_r_j, out_j)
    one(summ_u, cnt_u, x_u, w_l_u, b_u, w_r_u, out_u)


def _tc_sage(summ_j, cnt_j, x_j, wl_j, b_j, wr_j,
             summ_u, cnt_u, x_u, wl_u, b_u, wr_u, *, relu, normalize):
    body = functools.partial(_sage_body, relu=relu, normalize=normalize)
    return pl.pallas_call(
        body,
        out_shape=[jax.ShapeDtypeStruct((NP, D), _f32),
                   jax.ShapeDtypeStruct((NP, D), _f32)],
    )(summ_j, cnt_j, x_j, wl_j, b_j, wr_j,
      summ_u, cnt_u, x_u, wl_u, b_u, wr_u)


def _dot_body(a_ref, b_ref, o_ref):
    o_ref[...] = jnp.sum(a_ref[...] * b_ref[...], axis=1)


_TC3_BLK = 8192


def _tc_rowdot(a, b):
    grid = L // _TC3_BLK
    return pl.pallas_call(
        _dot_body,
        grid=(grid,),
        in_specs=[pl.BlockSpec((_TC3_BLK, D), lambda i: (i, 0)),
                  pl.BlockSpec((_TC3_BLK, D), lambda i: (i, 0))],
        out_specs=pl.BlockSpec((_TC3_BLK,), lambda i: (i,)),
        out_shape=jax.ShapeDtypeStruct((L,), _f32),
    )(a, b)


def kernel(x_user, x_job, edge_index_u2j, edge_index_j2u, edge_label_index,
           W1l_u2j, W1r_u2j, b1_u2j, W1l_j2u, W1r_j2u, b1_j2u,
           W2l_u2j, W2r_u2j, b2_u2j, W2l_j2u, W2r_j2u, b2_j2u):
    su2j = edge_index_u2j[0].astype(_i32)
    du2j = edge_index_u2j[1].astype(_i32)
    sj2u = edge_index_j2u[0].astype(_i32)
    dj2u = edge_index_j2u[1].astype(_i32)
    la = edge_label_index[0].astype(_i32)
    lb = edge_label_index[1].astype(_i32)
    xu = jnp.pad(x_user, ((0, NP - N), (0, 0)))
    xj = jnp.pad(x_job, ((0, NP - N), (0, 0)))
    znd = jnp.zeros((NP, D), _f32)
    ones = jnp.ones((CH, D), _f32)
    b1j = b1_u2j.reshape(1, D)
    b1u = b1_j2u.reshape(1, D)
    b2j = b2_u2j.reshape(1, D)
    b2u = b2_j2u.reshape(1, D)

    summ_j, summ_u, cnt_j, cnt_u = _get_agg(True)(
        xu, xj, su2j, du2j, sj2u, dj2u, znd, ones)
    h_job, h_user = _tc_sage(
        summ_j, cnt_j, xj, W1l_u2j, b1j, W1r_u2j,
        summ_u, cnt_u, xu, W1l_j2u, b1u, W1r_j2u,
        relu=True, normalize=False)

    summ2_j, summ2_u = _get_agg(False)(
        h_user, h_job, su2j, du2j, sj2u, dj2u, znd)
    zn_job, zn_user = _tc_sage(
        summ2_j, cnt_j, h_job, W2l_u2j, b2j, W2r_u2j,
        summ2_u, cnt_u, h_user, W2l_j2u, b2u, W2r_j2u,
        relu=False, normalize=True)

    ga, gb = _get_gather()(zn_user, zn_job, la, lb)
    return _tc_rowdot(ga, gb)


# speedup vs baseline: 5.1944x; 1.2733x over previous
import functools

import jax
import jax.numpy as jnp
from jax import lax
from jax.experimental import pallas as pl
from jax.experimental.pallas import tpu as pltpu
from jax.experimental.pallas import tpu_sc as plsc

N = 5000
NP = 5120
D = 128
E = 320000
L = 65536
NC = 2
NS = 16
RPT = NP // NS
EPT = E // NS
CH = 128
ITERS = 158
EPTB = ITERS * CH
SINK = N
LPW = L // (NC * NS)
CHL = 128
LITERS = LPW // CHL

_f32 = jnp.float32
_i32 = jnp.int32


def _mesh():
    return plsc.VectorSubcoreMesh(core_axis_name="c", subcore_axis_name="s")


def _make_agg(with_cnt):
    out_type = [jax.ShapeDtypeStruct((NP, D), _f32),
                jax.ShapeDtypeStruct((NP, D), _f32)]
    scratch = [
        pltpu.VMEM_SHARED((NP, D), _f32),
        pltpu.VMEM((ITERS, CH), _i32),
        pltpu.VMEM((ITERS, CH), _i32),
        pltpu.VMEM((CH, D), _f32),
        pltpu.VMEM((CH, D), _f32),
        pltpu.SemaphoreType.DMA,
        pltpu.SemaphoreType.DMA,
    ]
    if with_cnt:
        out_type += [jax.ShapeDtypeStruct((NP, D), _f32),
                     jax.ShapeDtypeStruct((NP, D), _f32)]
        scratch += [
            pltpu.VMEM((CH, D), _f32),
        ]

    @functools.partial(pl.kernel, mesh=_mesh(), out_type=out_type,
                       scratch_types=scratch)
    def agg(*refs):
        if with_cnt:
            (xu, xj, su, du, sj, dj, znd, ones,
             out_j, out_u, cnt_j, cnt_u,
             acc, sidx, didx, rbuf0, rbuf1, sem0, sem1,
             obuf) = refs
        else:
            (xu, xj, su, du, sj, dj, znd,
             out_j, out_u,
             acc, sidx, didx, rbuf0, rbuf1, sem0, sem1) = refs
        cid = lax.axis_index("c")
        sid = lax.axis_index("s")
        r0 = sid * RPT

        def run(x_hbm, src_hbm, dst_hbm, out_hbm, cnt_hbm):
            pltpu.sync_copy(src_hbm.at[sid], sidx)
            pltpu.sync_copy(dst_hbm.at[sid], didx)
            pltpu.sync_copy(znd.at[pl.ds(r0, RPT)], acc.at[pl.ds(r0, RPT)])
            if with_cnt:
                pltpu.sync_copy(ones, obuf)
            plsc.subcore_barrier()

            pltpu.async_copy(x_hbm.at[sidx.at[0]], rbuf0, sem0)
            pltpu.async_copy(x_hbm.at[sidx.at[1]], rbuf1, sem1)

            def step(k, carry):
                i0 = 2 * k
                i1 = 2 * k + 1
                pltpu.make_async_copy(x_hbm.at[sidx.at[i0]], rbuf0,
                                      sem0).wait()
                pltpu.sync_copy(rbuf0, acc.at[didx.at[i0]], add=True)

                @pl.when(i0 + 2 < ITERS)
                def _():
                    pltpu.async_copy(x_hbm.at[sidx.at[i0 + 2]], rbuf0, sem0)

                pltpu.make_async_copy(x_hbm.at[sidx.at[i1]], rbuf1,
                                      sem1).wait()
                pltpu.sync_copy(rbuf1, acc.at[didx.at[i1]], add=True)

                @pl.when(i1 + 2 < ITERS)
                def _():
                    pltpu.async_copy(x_hbm.at[sidx.at[i1 + 2]], rbuf1, sem1)

                return carry

            lax.fori_loop(0, ITERS // 2, step, 0)
            plsc.subcore_barrier()

            pltpu.sync_copy(acc.at[pl.ds(r0, RPT)], out_hbm.at[pl.ds(r0, RPT)])
            if with_cnt:
                pltpu.sync_copy(znd.at[pl.ds(r0, RPT)],
                                acc.at[pl.ds(r0, RPT)])
                plsc.subcore_barrier()

                def cstep(i, carry):
                    pltpu.sync_copy(obuf, acc.at[didx.at[i]], add=True)
                    return carry

                lax.fori_loop(0, ITERS, cstep, 0)
                plsc.subcore_barrier()
                pltpu.sync_copy(acc.at[pl.ds(r0, RPT)],
                                cnt_hbm.at[pl.ds(r0, RPT)])

        @pl.when(cid == 0)
        def _():
            run(xu, su, du, out_j, cnt_j if with_cnt else None)

        @pl.when(cid == 1)
        def _():
            run(xj, sj, dj, out_u, cnt_u if with_cnt else None)

    return agg


@functools.lru_cache(maxsize=None)
def _get_agg(with_cnt):
    return _make_agg(with_cnt)


@functools.lru_cache(maxsize=None)
def _get_gather():
    @functools.partial(
        pl.kernel, mesh=_mesh(),
        out_type=[jax.ShapeDtypeStruct((L, D), _f32),
                  jax.ShapeDtypeStruct((L, D), _f32)],
        scratch_types=[
            pltpu.VMEM((LITERS + 1, CHL), _i32),
            pltpu.VMEM((LITERS + 1, CHL), _i32),
            pltpu.VMEM((CHL, D), _f32),
            pltpu.VMEM((CHL, D), _f32),
            pltpu.SemaphoreType.DMA,
            pltpu.SemaphoreType.DMA,
        ])
    def _gather_pairs(zu, zj, la, lb, out_a, out_b,
                      iabuf, ibbuf, ra, rb, sema, semb):
        cid = lax.axis_index("c")
        sid = lax.axis_index("s")
        w = cid * NS + sid
        base = w * LPW
        pltpu.sync_copy(la.at[w], iabuf)
        pltpu.sync_copy(lb.at[w], ibbuf)
        pltpu.async_copy(zu.at[iabuf.at[0]], ra, sema)
        pltpu.async_copy(zj.at[ibbuf.at[0]], rb, semb)

        def step(i, carry):
            b = base + i * CHL
            pltpu.make_async_copy(zu.at[iabuf.at[i]], ra, sema).wait()
            pltpu.sync_copy(ra, out_a.at[pl.ds(b, CHL)])
            pltpu.async_copy(zu.at[iabuf.at[i + 1]], ra, sema)
            pltpu.make_async_copy(zj.at[ibbuf.at[i]], rb, semb).wait()
            pltpu.sync_copy(rb, out_b.at[pl.ds(b, CHL)])
            pltpu.async_copy(zj.at[ibbuf.at[i + 1]], rb, semb)
            return carry

        lax.fori_loop(0, LITERS, step, 0)
        pltpu.make_async_copy(zu.at[iabuf.at[LITERS]], ra, sema).wait()
        pltpu.make_async_copy(zj.at[ibbuf.at[LITERS]], rb, semb).wait()

    return _gather_pairs


def _mm_t(x, w):
    return lax.dot_general(x, w, (((1,), (1,)), ((), ())),
                           preferred_element_type=_f32)


def _sage_body(summ_j, cnt_j, x_j, w_l_j, b_j, w_r_j,
               summ_u, cnt_u, x_u, w_l_u, b_u, w_r_u,
               out_j, out_u, *, relu, normalize):
    def one(summ, cnt, x, wl, b, wr, out):
        mean = summ[...] / jnp.clip(cnt[...][:, 0:1], 1.0, None)
        h = _mm_t(mean, wl[...]) + b[...] + _mm_t(x[...], wr[...])
        if relu:
            h = jnp.maximum(h, 0.0)
        if normalize:
            nrm = jnp.sqrt(jnp.sum(h * h, axis=1, keepdims=True))
            h = h / jnp.clip(nrm, 1e-8, None)
        out[...] = h

    one(summ_j, cnt_j, x_j, w_l_j, b_j, w_r_j, out_j)
    one(summ_u, cnt_u, x_u, w_l_u, b_u, w_r_u, out_u)


def _tc_sage(summ_j, cnt_j, x_j, wl_j, b_j, wr_j,
             summ_u, cnt_u, x_u, wl_u, b_u, wr_u, *, relu, normalize):
    body = functools.partial(_sage_body, relu=relu, normalize=normalize)
    return pl.pallas_call(
        body,
        out_shape=[jax.ShapeDtypeStruct((NP, D), _f32),
                   jax.ShapeDtypeStruct((NP, D), _f32)],
    )(summ_j, cnt_j, x_j, wl_j, b_j, wr_j,
      summ_u, cnt_u, x_u, wl_u, b_u, wr_u)


def _dot_body(a_ref, b_ref, o_ref):
    o_ref[...] = jnp.sum(a_ref[...] * b_ref[...], axis=1)


_TC3_BLK = 8192


def _tc_rowdot(a, b):
    grid = L // _TC3_BLK
    return pl.pallas_call(
        _dot_body,
        grid=(grid,),
        in_specs=[pl.BlockSpec((_TC3_BLK, D), lambda i: (i, 0)),
                  pl.BlockSpec((_TC3_BLK, D), lambda i: (i, 0))],
        out_specs=pl.BlockSpec((_TC3_BLK,), lambda i: (i,)),
        out_shape=jax.ShapeDtypeStruct((L,), _f32),
    )(a, b)


def kernel(x_user, x_job, edge_index_u2j, edge_index_j2u, edge_label_index,
           W1l_u2j, W1r_u2j, b1_u2j, W1l_j2u, W1r_j2u, b1_j2u,
           W2l_u2j, W2r_u2j, b2_u2j, W2l_j2u, W2r_j2u, b2_j2u):
    def pack_edges(v, fill):
        v2 = v.astype(_i32).reshape(NS, EPT)
        v2 = jnp.pad(v2, ((0, 0), (0, EPTB - EPT)), constant_values=fill)
        return v2.reshape(NS, ITERS, CH)

    def pack_labels(v):
        v2 = v.astype(_i32).reshape(NC * NS, LITERS, CHL)
        return jnp.pad(v2, ((0, 0), (0, 1), (0, 0)))

    su2j = pack_edges(edge_index_u2j[0], 0)
    du2j = pack_edges(edge_index_u2j[1], SINK)
    sj2u = pack_edges(edge_index_j2u[0], 0)
    dj2u = pack_edges(edge_index_j2u[1], SINK)
    la = pack_labels(edge_label_index[0])
    lb = pack_labels(edge_label_index[1])
    xu = jnp.pad(x_user, ((0, NP - N), (0, 0)))
    xj = jnp.pad(x_job, ((0, NP - N), (0, 0)))
    znd = jnp.zeros((NP, D), _f32)
    ones = jnp.ones((CH, D), _f32)
    b1j = b1_u2j.reshape(1, D)
    b1u = b1_j2u.reshape(1, D)
    b2j = b2_u2j.reshape(1, D)
    b2u = b2_j2u.reshape(1, D)

    summ_j, summ_u, cnt_j, cnt_u = _get_agg(True)(
        xu, xj, su2j, du2j, sj2u, dj2u, znd, ones)
    h_job, h_user = _tc_sage(
        summ_j, cnt_j, xj, W1l_u2j, b1j, W1r_u2j,
        summ_u, cnt_u, xu, W1l_j2u, b1u, W1r_j2u,
        relu=True, normalize=False)

    summ2_j, summ2_u = _get_agg(False)(
        h_user, h_job, su2j, du2j, sj2u, dj2u, znd)
    zn_job, zn_user = _tc_sage(
        summ2_j, cnt_j, h_job, W2l_u2j, b2j, W2r_u2j,
        summ2_u, cnt_u, h_user, W2l_j2u, b2u, W2r_j2u,
        relu=False, normalize=True)

    ga, gb = _get_gather()(zn_user, zn_job, la, lb)
    return _tc_rowdot(ga, gb)


# speedup vs baseline: 5.5059x; 1.0600x over previous
import functools

import jax
import jax.numpy as jnp
from jax import lax
from jax.experimental import pallas as pl
from jax.experimental.pallas import tpu as pltpu
from jax.experimental.pallas import tpu_sc as plsc

N = 5000
NP = 5120
D = 128
E = 320000
L = 65536
NC = 2
NS = 16
RPT = NP // NS
EPT = E // NS
CH = 128
ITERS = 158
EPTB = ITERS * CH
SINK = N
CQ = 8
LPW = L // (NC * NS)
CHL = 128
LITERS = LPW // CHL

_f32 = jnp.float32
_i32 = jnp.int32


def _mesh():
    return plsc.VectorSubcoreMesh(core_axis_name="c", subcore_axis_name="s")


def _make_agg(with_cnt):
    out_type = [jax.ShapeDtypeStruct((NP, D), _f32),
                jax.ShapeDtypeStruct((NP, D), _f32)]
    scratch = [
        pltpu.VMEM_SHARED((NP, D), _f32),
        pltpu.VMEM((ITERS, CH), _i32),
        pltpu.VMEM((ITERS, CH), _i32),
        pltpu.VMEM((CH, D), _f32),
        pltpu.VMEM((CH, D), _f32),
        pltpu.SemaphoreType.DMA,
        pltpu.SemaphoreType.DMA,
        pltpu.SemaphoreType.DMA,
        pltpu.SemaphoreType.DMA,
    ]
    if with_cnt:
        out_type += [jax.ShapeDtypeStruct((NP, D), _f32),
                     jax.ShapeDtypeStruct((NP, D), _f32)]
        scratch += [
            pltpu.VMEM((CH, D), _f32),
        ]

    @functools.partial(pl.kernel, mesh=_mesh(), out_type=out_type,
                       scratch_types=scratch)
    def agg(*refs):
        if with_cnt:
            (xu, xj, su, du, sj, dj, znd, ones,
             out_j, out_u, cnt_j, cnt_u,
             acc, sidx, didx, rbuf0, rbuf1, sem0, sem1, sems0, sems1,
             obuf) = refs
        else:
            (xu, xj, su, du, sj, dj, znd,
             out_j, out_u,
             acc, sidx, didx, rbuf0, rbuf1, sem0, sem1,
             sems0, sems1) = refs
        cid = lax.axis_index("c")
        sid = lax.axis_index("s")
        r0 = sid * RPT

        def run(x_hbm, src_hbm, dst_hbm, out_hbm, cnt_hbm):
            pltpu.sync_copy(src_hbm.at[sid], sidx)
            pltpu.sync_copy(dst_hbm.at[sid], didx)
            pltpu.sync_copy(znd.at[pl.ds(r0, RPT)], acc.at[pl.ds(r0, RPT)])
            if with_cnt:
                pltpu.sync_copy(ones, obuf)
            plsc.subcore_barrier()

            pltpu.async_copy(x_hbm.at[sidx.at[0]], rbuf0, sem0)
            pltpu.async_copy(x_hbm.at[sidx.at[1]], rbuf1, sem1)

            def step(k, carry):
                i0 = 2 * k
                i1 = 2 * k + 1
                pltpu.make_async_copy(x_hbm.at[sidx.at[i0]], rbuf0,
                                      sem0).wait()
                pltpu.async_copy(rbuf0, acc.at[didx.at[i0]], sems0,
                                 add=True)
                pltpu.make_async_copy(x_hbm.at[sidx.at[i1]], rbuf1,
                                      sem1).wait()
                pltpu.async_copy(rbuf1, acc.at[didx.at[i1]], sems1,
                                 add=True)
                pltpu.make_async_copy(rbuf0, acc.at[didx.at[i0]],
                                      sems0).wait()

                @pl.when(i0 + 2 < ITERS)
                def _():
                    pltpu.async_copy(x_hbm.at[sidx.at[i0 + 2]], rbuf0, sem0)

                pltpu.make_async_copy(rbuf1, acc.at[didx.at[i1]],
                                      sems1).wait()

                @pl.when(i1 + 2 < ITERS)
                def _():
                    pltpu.async_copy(x_hbm.at[sidx.at[i1 + 2]], rbuf1, sem1)

                return carry

            lax.fori_loop(0, ITERS // 2, step, 0)
            plsc.subcore_barrier()

            pltpu.sync_copy(acc.at[pl.ds(r0, RPT)], out_hbm.at[pl.ds(r0, RPT)])
            if with_cnt:
                pltpu.sync_copy(znd.at[pl.ds(r0, RPT)],
                                acc.at[pl.ds(r0, RPT)])
                plsc.subcore_barrier()

                def cstep(i, carry):
                    @pl.when(i >= CQ)
                    def _():
                        pltpu.make_async_copy(obuf, acc.at[didx.at[i]],
                                              sems0).wait()
                    pltpu.async_copy(obuf, acc.at[didx.at[i]], sems0,
                                     add=True)
                    return carry

                lax.fori_loop(0, ITERS, cstep, 0)
                for j in range(CQ):
                    pltpu.make_async_copy(obuf, acc.at[didx.at[ITERS - CQ + j]],
                                          sems0).wait()
                plsc.subcore_barrier()
                pltpu.sync_copy(acc.at[pl.ds(r0, RPT)],
                                cnt_hbm.at[pl.ds(r0, RPT)])

        @pl.when(cid == 0)
        def _():
            run(xu, su, du, out_j, cnt_j if with_cnt else None)

        @pl.when(cid == 1)
        def _():
            run(xj, sj, dj, out_u, cnt_u if with_cnt else None)

    return agg


@functools.lru_cache(maxsize=None)
def _get_agg(with_cnt):
    return _make_agg(with_cnt)


@functools.lru_cache(maxsize=None)
def _get_gather():
    @functools.partial(
        pl.kernel, mesh=_mesh(),
        out_type=[jax.ShapeDtypeStruct((L, D), _f32),
                  jax.ShapeDtypeStruct((L, D), _f32)],
        scratch_types=[
            pltpu.VMEM((CHL,), _i32),
            pltpu.VMEM((CHL, D), _f32),
            pltpu.SemaphoreType.DMA,
        ])
    def _gather_pairs(zu, zj, la, lb, out_a, out_b, ibuf, rbuf, sem):
        cid = lax.axis_index("c")
        sid = lax.axis_index("s")
        base = (cid * NS + sid) * LPW

        def step(i, carry):
            b = base + i * CHL
            pltpu.sync_copy(la.at[pl.ds(b, CHL)], ibuf)
            pltpu.async_copy(zu.at[ibuf], rbuf, sem).wait()
            pltpu.sync_copy(rbuf, out_a.at[pl.ds(b, CHL)])
            pltpu.sync_copy(lb.at[pl.ds(b, CHL)], ibuf)
            pltpu.async_copy(zj.at[ibuf], rbuf, sem).wait()
            pltpu.sync_copy(rbuf, out_b.at[pl.ds(b, CHL)])
            return carry

        lax.fori_loop(0, LITERS, step, 0)

    return _gather_pairs


def _mm_t(x, w):
    return lax.dot_general(x, w, (((1,), (1,)), ((), ())),
                           preferred_element_type=_f32)


def _sage_body(summ_j, cnt_j, x_j, w_l_j, b_j, w_r_j,
               summ_u, cnt_u, x_u, w_l_u, b_u, w_r_u,
               out_j, out_u, *, relu, normalize):
    def one(summ, cnt, x, wl, b, wr, out):
        mean = summ[...] / jnp.clip(cnt[...][:, 0:1], 1.0, None)
        h = _mm_t(mean, wl[...]) + b[...] + _mm_t(x[...], wr[...])
        if relu:
            h = jnp.maximum(h, 0.0)
        if normalize:
            nrm = jnp.sqrt(jnp.sum(h * h, axis=1, keepdims=True))
            h = h / jnp.clip(nrm, 1e-8, None)
        out[...] = h

    one(summ_j, cnt_j, x_j, w_l_j, b_j, w_r_j, out_j)
    one(summ_u, cnt_u, x_u, w_l_u, b_u, w_r_u, out_u)


def _tc_sage(summ_j, cnt_j, x_j, wl_j, b_j, wr_j,
             summ_u, cnt_u, x_u, wl_u, b_u, wr_u, *, relu, normalize):
    body = functools.partial(_sage_body, relu=relu, normalize=normalize)
    return pl.pallas_call(
        body,
        out_shape=[jax.ShapeDtypeStruct((NP, D), _f32),
                   jax.ShapeDtypeStruct((NP, D), _f32)],
    )(summ_j, cnt_j, x_j, wl_j, b_j, wr_j,
      summ_u, cnt_u, x_u, wl_u, b_u, wr_u)


def _dot_body(a_ref, b_ref, o_ref):
    o_ref[...] = jnp.sum(a_ref[...] * b_ref[...], axis=1)


_TC3_BLK = 8192


def _tc_rowdot(a, b):
    grid = L // _TC3_BLK
    return pl.pallas_call(
        _dot_body,
        grid=(grid,),
        in_specs=[pl.BlockSpec((_TC3_BLK, D), lambda i: (i, 0)),
                  pl.BlockSpec((_TC3_BLK, D), lambda i: (i, 0))],
        out_specs=pl.BlockSpec((_TC3_BLK,), lambda i: (i,)),
        out_shape=jax.ShapeDtypeStruct((L,), _f32),
    )(a, b)


def kernel(x_user, x_job, edge_index_u2j, edge_index_j2u, edge_label_index,
           W1l_u2j, W1r_u2j, b1_u2j, W1l_j2u, W1r_j2u, b1_j2u,
           W2l_u2j, W2r_u2j, b2_u2j, W2l_j2u, W2r_j2u, b2_j2u):
    def pack_edges(v, fill):
        v2 = v.astype(_i32).reshape(NS, EPT)
        v2 = jnp.pad(v2, ((0, 0), (0, EPTB - EPT)), constant_values=fill)
        return v2.reshape(NS, ITERS, CH)

    su2j = pack_edges(edge_index_u2j[0], 0)
    du2j = pack_edges(edge_index_u2j[1], SINK)
    sj2u = pack_edges(edge_index_j2u[0], 0)
    dj2u = pack_edges(edge_index_j2u[1], SINK)
    la = edge_label_index[0].astype(_i32)
    lb = edge_label_index[1].astype(_i32)
    xu = jnp.pad(x_user, ((0, NP - N), (0, 0)))
    xj = jnp.pad(x_job, ((0, NP - N), (0, 0)))
    znd = jnp.zeros((NP, D), _f32)
    ones = jnp.ones((CH, D), _f32)
    b1j = b1_u2j.reshape(1, D)
    b1u = b1_j2u.reshape(1, D)
    b2j = b2_u2j.reshape(1, D)
    b2u = b2_j2u.reshape(1, D)

    summ_j, summ_u, cnt_j, cnt_u = _get_agg(True)(
        xu, xj, su2j, du2j, sj2u, dj2u, znd, ones)
    h_job, h_user = _tc_sage(
        summ_j, cnt_j, xj, W1l_u2j, b1j, W1r_u2j,
        summ_u, cnt_u, xu, W1l_j2u, b1u, W1r_j2u,
        relu=True, normalize=False)

    summ2_j, summ2_u = _get_agg(False)(
        h_user, h_job, su2j, du2j, sj2u, dj2u, znd)
    zn_job, zn_user = _tc_sage(
        summ2_j, cnt_j, h_job, W2l_u2j, b2j, W2r_u2j,
        summ2_u, cnt_u, h_user, W2l_j2u, b2u, W2r_j2u,
        relu=False, normalize=True)

    ga, gb = _get_gather()(zn_user, zn_job, la, lb)
    return _tc_rowdot(ga, gb)


# speedup vs baseline: 6.3443x; 1.1523x over previous
import functools

import jax
import jax.numpy as jnp
from jax import lax
from jax.experimental import pallas as pl
from jax.experimental.pallas import tpu as pltpu
from jax.experimental.pallas import tpu_sc as plsc

N = 5000
NP = 5120
D = 128
E = 320000
L = 65536
NC = 2
NS = 16
RPT = NP // NS
EPT = E // NS
CH = 128
ITERS = 158
EPTB = ITERS * CH
SINK = N
CQ = 8
LPW = L // (NC * NS)
CHL = 128
LITERS = LPW // CHL

_f32 = jnp.float32
_i32 = jnp.int32


def _mesh():
    return plsc.VectorSubcoreMesh(core_axis_name="c", subcore_axis_name="s")


def _make_agg(with_cnt):
    out_type = [jax.ShapeDtypeStruct((NP, D), _f32),
                jax.ShapeDtypeStruct((NP, D), _f32)]
    scratch = [
        pltpu.VMEM_SHARED((NP, D), _f32),
        pltpu.VMEM((ITERS, CH), _i32),
        pltpu.VMEM((ITERS, CH), _i32),
        pltpu.VMEM((CH, D), _f32),
        pltpu.VMEM((CH, D), _f32),
        pltpu.SemaphoreType.DMA,
        pltpu.SemaphoreType.DMA,
        pltpu.SemaphoreType.DMA,
        pltpu.SemaphoreType.DMA,
    ]
    if with_cnt:
        out_type += [jax.ShapeDtypeStruct((NP, D), _f32),
                     jax.ShapeDtypeStruct((NP, D), _f32)]
        scratch += [
            pltpu.VMEM((CH, D), _f32),
        ]

    @functools.partial(pl.kernel, mesh=_mesh(), out_type=out_type,
                       scratch_types=scratch)
    def agg(*refs):
        if with_cnt:
            (xu, xj, su, du, sj, dj, znd, ones,
             out_j, out_u, cnt_j, cnt_u,
             acc, sidx, didx, rbuf0, rbuf1, sem0, sem1, sems0, sems1,
             obuf) = refs
        else:
            (xu, xj, su, du, sj, dj, znd,
             out_j, out_u,
             acc, sidx, didx, rbuf0, rbuf1, sem0, sem1,
             sems0, sems1) = refs
        cid = lax.axis_index("c")
        sid = lax.axis_index("s")
        r0 = sid * RPT

        def run(x_hbm, src_hbm, dst_hbm, out_hbm, cnt_hbm):
            pltpu.sync_copy(src_hbm.at[sid], sidx)
            pltpu.sync_copy(dst_hbm.at[sid], didx)
            pltpu.sync_copy(znd.at[pl.ds(r0, RPT)], acc.at[pl.ds(r0, RPT)])
            if with_cnt:
                pltpu.sync_copy(ones, obuf)
            plsc.subcore_barrier()

            pltpu.async_copy(x_hbm.at[sidx.at[0]], rbuf0, sem0)
            pltpu.async_copy(x_hbm.at[sidx.at[1]], rbuf1, sem1)

            def step(k, carry):
                i0 = 2 * k
                i1 = 2 * k + 1
                pltpu.make_async_copy(x_hbm.at[sidx.at[i0]], rbuf0,
                                      sem0).wait()
                pltpu.sync_copy(rbuf0, acc.at[didx.at[i0]], add=True)

                @pl.when(i0 + 2 < ITERS)
                def _():
                    pltpu.async_copy(x_hbm.at[sidx.at[i0 + 2]], rbuf0, sem0)

                pltpu.make_async_copy(x_hbm.at[sidx.at[i1]], rbuf1,
                                      sem1).wait()
                pltpu.sync_copy(rbuf1, acc.at[didx.at[i1]], add=True)

                @pl.when(i1 + 2 < ITERS)
                def _():
                    pltpu.async_copy(x_hbm.at[sidx.at[i1 + 2]], rbuf1, sem1)

                return carry

            lax.fori_loop(0, ITERS // 2, step, 0)
            plsc.subcore_barrier()

            pltpu.sync_copy(acc.at[pl.ds(r0, RPT)], out_hbm.at[pl.ds(r0, RPT)])
            if with_cnt:
                pltpu.sync_copy(znd.at[pl.ds(r0, RPT)],
                                acc.at[pl.ds(r0, RPT)])
                plsc.subcore_barrier()

                def cstep(i, carry):
                    pltpu.sync_copy(obuf, acc.at[didx.at[i]], add=True)
                    return carry

                lax.fori_loop(0, ITERS, cstep, 0)
                plsc.subcore_barrier()
                pltpu.sync_copy(acc.at[pl.ds(r0, RPT)],
                                cnt_hbm.at[pl.ds(r0, RPT)])

        @pl.when(cid == 0)
        def _():
            run(xu, su, du, out_j, cnt_j if with_cnt else None)

        @pl.when(cid == 1)
        def _():
            run(xj, sj, dj, out_u, cnt_u if with_cnt else None)

    return agg


@functools.lru_cache(maxsize=None)
def _get_agg(with_cnt):
    return _make_agg(with_cnt)


@functools.lru_cache(maxsize=None)
def _get_gather():
    @functools.partial(
        pl.kernel, mesh=_mesh(),
        out_type=[jax.ShapeDtypeStruct((L, D), _f32),
                  jax.ShapeDtypeStruct((L, D), _f32)],
        scratch_types=[
            pltpu.VMEM((CHL,), _i32),
            pltpu.VMEM((CHL, D), _f32),
            pltpu.SemaphoreType.DMA,
        ])
    def _gather_pairs(zu, zj, la, lb, out_a, out_b, ibuf, rbuf, sem):
        cid = lax.axis_index("c")
        sid = lax.axis_index("s")
        base = (cid * NS + sid) * LPW

        def step(i, carry):
            b = base + i * CHL
            pltpu.sync_copy(la.at[pl.ds(b, CHL)], ibuf)
            pltpu.async_copy(zu.at[ibuf], rbuf, sem).wait()
            pltpu.sync_copy(rbuf, out_a.at[pl.ds(b, CHL)])
            pltpu.sync_copy(lb.at[pl.ds(b, CHL)], ibuf)
            pltpu.async_copy(zj.at[ibuf], rbuf, sem).wait()
            pltpu.sync_copy(rbuf, out_b.at[pl.ds(b, CHL)])
            return carry

        lax.fori_loop(0, LITERS, step, 0)

    return _gather_pairs


def _mm_t(x, w):
    return lax.dot_general(x, w, (((1,), (1,)), ((), ())),
                           preferred_element_type=_f32)


def _sage_body(summ_j, cnt_j, x_j, w_l_j, b_j, w_r_j,
               summ_u, cnt_u, x_u, w_l_u, b_u, w_r_u,
               out_j, out_u, *, relu, normalize):
    def one(summ, cnt, x, wl, b, wr, out):
        mean = summ[...] / jnp.clip(cnt[...][:, 0:1], 1.0, None)
        h = _mm_t(mean, wl[...]) + b[...] + _mm_t(x[...], wr[...])
        if relu:
            h = jnp.maximum(h, 0.0)
        if normalize:
            nrm = jnp.sqrt(jnp.sum(h * h, axis=1, keepdims=True))
            h = h / jnp.clip(nrm, 1e-8, None)
        out[...] = h

    one(summ_j, cnt_j, x_j, w_l_j, b_j, w_r_j, out_j)
    one(summ_u, cnt_u, x_u, w_l_u, b_u, w_r_u, out_u)


def _tc_sage(summ_j, cnt_j, x_j, wl_j, b_j, wr_j,
             summ_u, cnt_u, x_u, wl_u, b_u, wr_u, *, relu, normalize):
    body = functools.partial(_sage_body, relu=relu, normalize=normalize)
    return pl.pallas_call(
        body,
        out_shape=[jax.ShapeDtypeStruct((NP, D), _f32),
                   jax.ShapeDtypeStruct((NP, D), _f32)],
    )(summ_j, cnt_j, x_j, wl_j, b_j, wr_j,
      summ_u, cnt_u, x_u, wl_u, b_u, wr_u)


def _dot_body(a_ref, b_ref, o_ref):
    o_ref[...] = jnp.sum(a_ref[...] * b_ref[...], axis=1)


_TC3_BLK = 8192


def _tc_rowdot(a, b):
    grid = L // _TC3_BLK
    return pl.pallas_call(
        _dot_body,
        grid=(grid,),
        in_specs=[pl.BlockSpec((_TC3_BLK, D), lambda i: (i, 0)),
                  pl.BlockSpec((_TC3_BLK, D), lambda i: (i, 0))],
        out_specs=pl.BlockSpec((_TC3_BLK,), lambda i: (i,)),
        out_shape=jax.ShapeDtypeStruct((L,), _f32),
    )(a, b)


def kernel(x_user, x_job, edge_index_u2j, edge_index_j2u, edge_label_index,
           W1l_u2j, W1r_u2j, b1_u2j, W1l_j2u, W1r_j2u, b1_j2u,
           W2l_u2j, W2r_u2j, b2_u2j, W2l_j2u, W2r_j2u, b2_j2u):
    def pack_edges(v, fill):
        v2 = v.astype(_i32).reshape(NS, EPT)
        v2 = jnp.pad(v2, ((0, 0), (0, EPTB - EPT)), constant_values=fill)
        return v2.reshape(NS, ITERS, CH)

    su2j = pack_edges(edge_index_u2j[0], 0)
    du2j = pack_edges(edge_index_u2j[1], SINK)
    sj2u = pack_edges(edge_index_j2u[0], 0)
    dj2u = pack_edges(edge_index_j2u[1], SINK)
    la = edge_label_index[0].astype(_i32)
    lb = edge_label_index[1].astype(_i32)
    xu = jnp.pad(x_user, ((0, NP - N), (0, 0)))
    xj = jnp.pad(x_job, ((0, NP - N), (0, 0)))
    znd = jnp.zeros((NP, D), _f32)
    ones = jnp.ones((CH, D), _f32)
    b1j = b1_u2j.reshape(1, D)
    b1u = b1_j2u.reshape(1, D)
    b2j = b2_u2j.reshape(1, D)
    b2u = b2_j2u.reshape(1, D)

    summ_j, summ_u, cnt_j, cnt_u = _get_agg(True)(
        xu, xj, su2j, du2j, sj2u, dj2u, znd, ones)
    h_job, h_user = _tc_sage(
        summ_j, cnt_j, xj, W1l_u2j, b1j, W1r_u2j,
        summ_u, cnt_u, xu, W1l_j2u, b1u, W1r_j2u,
        relu=True, normalize=False)

    summ2_j, summ2_u = _get_agg(False)(
        h_user, h_job, su2j, du2j, sj2u, dj2u, znd)
    zn_job, zn_user = _tc_sage(
        summ2_j, cnt_j, h_job, W2l_u2j, b2j, W2r_u2j,
        summ2_u, cnt_u, h_user, W2l_j2u, b2u, W2r_j2u,
        relu=False, normalize=True)

    ga, gb = _get_gather()(zn_user, zn_job, la, lb)
    return _tc_rowdot(ga, gb)


# speedup vs baseline: 6.3467x; 1.0004x over previous
import functools

import jax
import jax.numpy as jnp
from jax import lax
from jax.experimental import pallas as pl
from jax.experimental.pallas import tpu as pltpu
from jax.experimental.pallas import tpu_sc as plsc

N = 5000
NP = 5120
D = 128
E = 320000
L = 65536
NC = 2
NS = 16
RPT = NP // NS
EPT = E // NS
CH = 128
ITERS = 158
EPTB = ITERS * CH
SINK = N
LPW = L // (NC * NS)
CHL = 128
LITERS = LPW // CHL

_f32 = jnp.float32
_i32 = jnp.int32


def _mesh():
    return plsc.VectorSubcoreMesh(core_axis_name="c", subcore_axis_name="s")


def _make_agg(with_cnt):
    out_type = [jax.ShapeDtypeStruct((NP, D), _f32),
                jax.ShapeDtypeStruct((NP, D), _f32)]
    scratch = [
        pltpu.VMEM_SHARED((NP, D), _f32),
        pltpu.VMEM((ITERS, CH), _i32),
        pltpu.VMEM((ITERS, CH), _i32),
        pltpu.VMEM((CH, D), _f32),
        pltpu.VMEM((CH, D), _f32),
        pltpu.SemaphoreType.DMA,
        pltpu.SemaphoreType.DMA,
    ]
    if with_cnt:
        out_type += [jax.ShapeDtypeStruct((NP, D), _f32),
                     jax.ShapeDtypeStruct((NP, D), _f32)]
        scratch += [
            pltpu.VMEM((CH, D), _f32),
        ]

    @functools.partial(pl.kernel, mesh=_mesh(), out_type=out_type,
                       scratch_types=scratch)
    def agg(*refs):
        if with_cnt:
            (xu, xj, su, du, sj, dj, znd, ones,
             out_j, out_u, cnt_j, cnt_u,
             acc, sidx, didx, rbuf0, rbuf1, sem0, sem1,
             obuf) = refs
        else:
            (xu, xj, su, du, sj, dj, znd,
             out_j, out_u,
             acc, sidx, didx, rbuf0, rbuf1, sem0, sem1) = refs
        cid = lax.axis_index("c")
        sid = lax.axis_index("s")
        r0 = sid * RPT

        def run(x_hbm, src_hbm, dst_hbm, out_hbm, cnt_hbm):
            pltpu.sync_copy(src_hbm.at[sid], sidx)
            pltpu.sync_copy(dst_hbm.at[sid], didx)
            pltpu.sync_copy(znd.at[pl.ds(r0, RPT)], acc.at[pl.ds(r0, RPT)])
            if with_cnt:
                pltpu.sync_copy(ones, obuf)
            plsc.subcore_barrier()

            pltpu.async_copy(x_hbm.at[sidx.at[0]], rbuf0, sem0)
            pltpu.async_copy(x_hbm.at[sidx.at[1]], rbuf1, sem1)

            def step(k, carry):
                i0 = 2 * k
                i1 = 2 * k + 1
                pltpu.make_async_copy(x_hbm.at[sidx.at[i0]], rbuf0,
                                      sem0).wait()
                pltpu.sync_copy(rbuf0, acc.at[didx.at[i0]], add=True)

                @pl.when(i0 + 2 < ITERS)
                def _():
                    pltpu.async_copy(x_hbm.at[sidx.at[i0 + 2]], rbuf0, sem0)

                pltpu.make_async_copy(x_hbm.at[sidx.at[i1]], rbuf1,
                                      sem1).wait()
                pltpu.sync_copy(rbuf1, acc.at[didx.at[i1]], add=True)

                @pl.when(i1 + 2 < ITERS)
                def _():
                    pltpu.async_copy(x_hbm.at[sidx.at[i1 + 2]], rbuf1, sem1)

                return carry

            lax.fori_loop(0, ITERS // 2, step, 0)
            plsc.subcore_barrier()

            pltpu.sync_copy(acc.at[pl.ds(r0, RPT)], out_hbm.at[pl.ds(r0, RPT)])
            if with_cnt:
                pltpu.sync_copy(znd.at[pl.ds(r0, RPT)],
                                acc.at[pl.ds(r0, RPT)])
                plsc.subcore_barrier()

                def cstep(i, carry):
                    pltpu.sync_copy(obuf, acc.at[didx.at[i]], add=True)
                    return carry

                lax.fori_loop(0, ITERS, cstep, 0)
                plsc.subcore_barrier()
                pltpu.sync_copy(acc.at[pl.ds(r0, RPT)],
                                cnt_hbm.at[pl.ds(r0, RPT)])

        @pl.when(cid == 0)
        def _():
            run(xu, su, du, out_j, cnt_j if with_cnt else None)

        @pl.when(cid == 1)
        def _():
            run(xj, sj, dj, out_u, cnt_u if with_cnt else None)

    return agg


@functools.lru_cache(maxsize=None)
def _get_agg(with_cnt):
    return _make_agg(with_cnt)


@functools.lru_cache(maxsize=None)
def _get_gather():
    @functools.partial(
        pl.kernel, mesh=_mesh(),
        out_type=[jax.ShapeDtypeStruct((L, D), _f32),
                  jax.ShapeDtypeStruct((L, D), _f32)],
        scratch_types=[
            pltpu.VMEM((CHL,), _i32),
            pltpu.VMEM((CHL, D), _f32),
            pltpu.SemaphoreType.DMA,
        ])
    def _gather_pairs(zu, zj, la, lb, out_a, out_b, ibuf, rbuf, sem):
        cid = lax.axis_index("c")
        sid = lax.axis_index("s")
        base = (cid * NS + sid) * LPW

        def step(i, carry):
            b = base + i * CHL
            pltpu.sync_copy(la.at[pl.ds(b, CHL)], ibuf)
            pltpu.async_copy(zu.at[ibuf], rbuf, sem).wait()
            pltpu.sync_copy(rbuf, out_a.at[pl.ds(b, CHL)])
            pltpu.sync_copy(lb.at[pl.ds(b, CHL)], ibuf)
            pltpu.async_copy(zj.at[ibuf], rbuf, sem).wait()
            pltpu.sync_copy(rbuf, out_b.at[pl.ds(b, CHL)])
            return carry

        lax.fori_loop(0, LITERS, step, 0)

    return _gather_pairs


def _mm_t(x, w):
    return lax.dot_general(x, w, (((1,), (1,)), ((), ())),
                           preferred_element_type=_f32)


def _sage_body(summ_j, cnt_j, x_j, w_l_j, b_j, w_r_j,
               summ_u, cnt_u, x_u, w_l_u, b_u, w_r_u,
               out_j, out_u, *, relu, normalize):
    def one(summ, cnt, x, wl, b, wr, out):
        mean = summ[...] / jnp.clip(cnt[...][:, 0:1], 1.0, None)
        h = _mm_t(mean, wl[...]) + b[...] + _mm_t(x[...], wr[...])
        if relu:
            h = jnp.maximum(h, 0.0)
        if normalize:
            nrm = jnp.sqrt(jnp.sum(h * h, axis=1, keepdims=True))
            h = h / jnp.clip(nrm, 1e-8, None)
        out[...] = h

    one(summ_j, cnt_j, x_j, w_l_j, b_j, w_r_j, out_j)
    one(summ_u, cnt_u, x_u, w_l_u, b_u, w_r_u, out_u)


def _tc_sage(summ_j, cnt_j, x_j, wl_j, b_j, wr_j,
             summ_u, cnt_u, x_u, wl_u, b_u, wr_u, *, relu, normalize):
    body = functools.partial(_sage_body, relu=relu, normalize=normalize)
    return pl.pallas_call(
        body,
        out_shape=[jax.ShapeDtypeStruct((NP, D), _f32),
                   jax.ShapeDtypeStruct((NP, D), _f32)],
    )(summ_j, cnt_j, x_j, wl_j, b_j, wr_j,
      summ_u, cnt_u, x_u, wl_u, b_u, wr_u)


def _dot_body(a_ref, b_ref, o_ref):
    o_ref[...] = jnp.sum(a_ref[...] * b_ref[...], axis=1)


_TC3_BLK = 8192


def _tc_rowdot(a, b):
    grid = L // _TC3_BLK
    return pl.pallas_call(
        _dot_body,
        grid=(grid,),
        in_specs=[pl.BlockSpec((_TC3_BLK, D), lambda i: (i, 0)),
                  pl.BlockSpec((_TC3_BLK, D), lambda i: (i, 0))],
        out_specs=pl.BlockSpec((_TC3_BLK,), lambda i: (i,)),
        out_shape=jax.ShapeDtypeStruct((L,), _f32),
    )(a, b)


def kernel(x_user, x_job, edge_index_u2j, edge_index_j2u, edge_label_index,
           W1l_u2j, W1r_u2j, b1_u2j, W1l_j2u, W1r_j2u, b1_j2u,
           W2l_u2j, W2r_u2j, b2_u2j, W2l_j2u, W2r_j2u, b2_j2u):
    def pack_edges(v, fill):
        v2 = v.astype(_i32).reshape(NS, EPT)
        v2 = jnp.pad(v2, ((0, 0), (0, EPTB - EPT)), constant_values=fill)
        return v2.reshape(NS, ITERS, CH)

    su2j = pack_edges(edge_index_u2j[0], 0)
    du2j = pack_edges(edge_index_u2j[1], SINK)
    sj2u = pack_edges(edge_index_j2u[0], 0)
    dj2u = pack_edges(edge_index_j2u[1], SINK)
    la = edge_label_index[0].astype(_i32)
    lb = edge_label_index[1].astype(_i32)
    xu = jnp.pad(x_user, ((0, NP - N), (0, 0)))
    xj = jnp.pad(x_job, ((0, NP - N), (0, 0)))
    znd = jnp.zeros((NP, D), _f32)
    ones = jnp.ones((CH, D), _f32)
    b1j = b1_u2j.reshape(1, D)
    b1u = b1_j2u.reshape(1, D)
    b2j = b2_u2j.reshape(1, D)
    b2u = b2_j2u.reshape(1, D)

    summ_j, summ_u, cnt_j, cnt_u = _get_agg(True)(
        xu, xj, su2j, du2j, sj2u, dj2u, znd, ones)
    h_job, h_user = _tc_sage(
        summ_j, cnt_j, xj, W1l_u2j, b1j, W1r_u2j,
        summ_u, cnt_u, xu, W1l_j2u, b1u, W1r_j2u,
        relu=True, normalize=False)

    summ2_j, summ2_u = _get_agg(False)(
        h_user, h_job, su2j, du2j, sj2u, dj2u, znd)
    zn_job, zn_user = _tc_sage(
        summ2_j, cnt_j, h_job, W2l_u2j, b2j, W2r_u2j,
        summ2_u, cnt_u, h_user, W2l_j2u, b2u, W2r_j2u,
        relu=False, normalize=True)

    ga, gb = _get_gather()(zn_user, zn_job, la, lb)
    return _tc_rowdot(ga, gb)


# speedup vs baseline: 6.5226x; 1.0277x over previous
import functools

import jax
import jax.numpy as jnp
from jax import lax
from jax.experimental import pallas as pl
from jax.experimental.pallas import tpu as pltpu
from jax.experimental.pallas import tpu_sc as plsc

N = 5000
NP = 5120
D = 128
E = 320000
L = 65536
NC = 2
NS = 16
RPT = NP // NS
EPT = E // NS
CH = 128
ITERS = 158
EPTB = ITERS * CH
SINK = N
LPW = L // (NC * NS)
CHL = 128
LITERS = LPW // CHL

_f32 = jnp.float32
_i32 = jnp.int32


def _mesh():
    return plsc.VectorSubcoreMesh(core_axis_name="c", subcore_axis_name="s")


def _make_agg(with_cnt):
    out_type = [jax.ShapeDtypeStruct((NP, D), _f32),
                jax.ShapeDtypeStruct((NP, D), _f32)]
    scratch = [
        pltpu.VMEM_SHARED((NP, D), _f32),
        pltpu.VMEM((ITERS, CH), _i32),
        pltpu.VMEM((ITERS, CH), _i32),
        pltpu.VMEM((CH, D), _f32),
        pltpu.VMEM((CH, D), _f32),
        pltpu.SemaphoreType.DMA,
        pltpu.SemaphoreType.DMA,
    ]
    if with_cnt:
        out_type += [jax.ShapeDtypeStruct((NP, D), _f32),
                     jax.ShapeDtypeStruct((NP, D), _f32)]
        scratch += [
            pltpu.VMEM((CH, D), _f32),
        ]

    @functools.partial(pl.kernel, mesh=_mesh(), out_type=out_type,
                       scratch_types=scratch)
    def agg(*refs):
        if with_cnt:
            (xu, xj, su, du, sj, dj, znd, ones,
             out_j, out_u, cnt_j, cnt_u,
             acc, sidx, didx, rbuf0, rbuf1, sem0, sem1,
             obuf) = refs
        else:
            (xu, xj, su, du, sj, dj, znd,
             out_j, out_u,
             acc, sidx, didx, rbuf0, rbuf1, sem0, sem1) = refs
        cid = lax.axis_index("c")
        sid = lax.axis_index("s")
        r0 = sid * RPT

        def run(x_hbm, src_hbm, dst_hbm, out_hbm, cnt_hbm):
            pltpu.sync_copy(src_hbm.at[sid], sidx)
            pltpu.sync_copy(dst_hbm.at[sid], didx)
            pltpu.sync_copy(znd.at[pl.ds(r0, RPT)], acc.at[pl.ds(r0, RPT)])
            if with_cnt:
                pltpu.sync_copy(ones, obuf)
            plsc.subcore_barrier()

            pltpu.async_copy(x_hbm.at[sidx.at[0]], rbuf0, sem0)
            pltpu.async_copy(x_hbm.at[sidx.at[1]], rbuf1, sem1)

            def step(k, carry):
                i0 = 2 * k
                i1 = 2 * k + 1
                pltpu.make_async_copy(x_hbm.at[sidx.at[i0]], rbuf0,
                                      sem0).wait()
                pltpu.sync_copy(rbuf0, acc.at[didx.at[i0]], add=True)

                @pl.when(i0 + 2 < ITERS)
                def _():
                    pltpu.async_copy(x_hbm.at[sidx.at[i0 + 2]], rbuf0, sem0)

                pltpu.make_async_copy(x_hbm.at[sidx.at[i1]], rbuf1,
                                      sem1).wait()
                pltpu.sync_copy(rbuf1, acc.at[didx.at[i1]], add=True)

                @pl.when(i1 + 2 < ITERS)
                def _():
                    pltpu.async_copy(x_hbm.at[sidx.at[i1 + 2]], rbuf1, sem1)

                return carry

            lax.fori_loop(0, ITERS // 2, step, 0)
            plsc.subcore_barrier()

            pltpu.sync_copy(acc.at[pl.ds(r0, RPT)], out_hbm.at[pl.ds(r0, RPT)])
            if with_cnt:
                pltpu.sync_copy(znd.at[pl.ds(r0, RPT)],
                                acc.at[pl.ds(r0, RPT)])
                plsc.subcore_barrier()

                def cstep(i, carry):
                    pltpu.sync_copy(obuf, acc.at[didx.at[i]], add=True)
                    return carry

                lax.fori_loop(0, ITERS, cstep, 0)
                plsc.subcore_barrier()
                pltpu.sync_copy(acc.at[pl.ds(r0, RPT)],
                                cnt_hbm.at[pl.ds(r0, RPT)])

        @pl.when(cid == 0)
        def _():
            run(xu, su, du, out_j, cnt_j if with_cnt else None)

        @pl.when(cid == 1)
        def _():
            run(xj, sj, dj, out_u, cnt_u if with_cnt else None)

    return agg


@functools.lru_cache(maxsize=None)
def _get_agg(with_cnt):
    return _make_agg(with_cnt)


@functools.lru_cache(maxsize=None)
def _get_gather():
    @functools.partial(
        pl.kernel, mesh=_mesh(),
        out_type=[jax.ShapeDtypeStruct((L, D), _f32),
                  jax.ShapeDtypeStruct((L, D), _f32)],
        scratch_types=[
            pltpu.VMEM((CHL,), _i32),
            pltpu.VMEM((CHL,), _i32),
            pltpu.VMEM((CHL, D), _f32),
            pltpu.VMEM((CHL, D), _f32),
            pltpu.SemaphoreType.DMA,
            pltpu.SemaphoreType.DMA,
        ])
    def _gather_pairs(zu, zj, la, lb, out_a, out_b,
                      ia, ib, ra, rb, sema, semb):
        cid = lax.axis_index("c")
        sid = lax.axis_index("s")
        base = (cid * NS + sid) * LPW

        def step(i, carry):
            b = base + i * CHL
            pltpu.sync_copy(la.at[pl.ds(b, CHL)], ia)
            pltpu.async_copy(zu.at[ia], ra, sema)
            pltpu.sync_copy(lb.at[pl.ds(b, CHL)], ib)
            pltpu.async_copy(zj.at[ib], rb, semb)
            pltpu.make_async_copy(zu.at[ia], ra, sema).wait()
            pltpu.sync_copy(ra, out_a.at[pl.ds(b, CHL)])
            pltpu.make_async_copy(zj.at[ib], rb, semb).wait()
            pltpu.sync_copy(rb, out_b.at[pl.ds(b, CHL)])
            return carry

        lax.fori_loop(0, LITERS, step, 0)

    return _gather_pairs


def _mm_t(x, w):
    return lax.dot_general(x, w, (((1,), (1,)), ((), ())),
                           preferred_element_type=_f32)


def _sage_body(summ_j, cnt_j, x_j, w_l_j, b_j, w_r_j,
               summ_u, cnt_u, x_u, w_l_u, b_u, w_r_u,
               out_j, out_u, *, relu, normalize):
    def one(summ, cnt, x, wl, b, wr, out):
        mean = summ[...] / jnp.clip(cnt[...][:, 0:1], 1.0, None)
        h = _mm_t(mean, wl[...]) + b[...] + _mm_t(x[...], wr[...])
        if relu:
            h = jnp.maximum(h, 0.0)
        if normalize:
            nrm = jnp.sqrt(jnp.sum(h * h, axis=1, keepdims=True))
            h = h / jnp.clip(nrm, 1e-8, None)
        out[...] = h

    one(summ_j, cnt_j, x_j, w_l_j, b_j, w_r_j, out_j)
    one(summ_u, cnt_u, x_u, w_l_u, b_u, w_r_u, out_u)


def _tc_sage(summ_j, cnt_j, x_j, wl_j, b_j, wr_j,
             summ_u, cnt_u, x_u, wl_u, b_u, wr_u, *, relu, normalize):
    body = functools.partial(_sage_body, relu=relu, normalize=normalize)
    return pl.pallas_call(
        body,
        out_shape=[jax.ShapeDtypeStruct((NP, D), _f32),
                   jax.ShapeDtypeStruct((NP, D), _f32)],
    )(summ_j, cnt_j, x_j, wl_j, b_j, wr_j,
      summ_u, cnt_u, x_u, wl_u, b_u, wr_u)


def _dot_body(a_ref, b_ref, o_ref):
    o_ref[...] = jnp.sum(a_ref[...] * b_ref[...], axis=1)


_TC3_BLK = 8192


def _tc_rowdot(a, b):
    grid = L // _TC3_BLK
    return pl.pallas_call(
        _dot_body,
        grid=(grid,),
        in_specs=[pl.BlockSpec((_TC3_BLK, D), lambda i: (i, 0)),
                  pl.BlockSpec((_TC3_BLK, D), lambda i: (i, 0))],
        out_specs=pl.BlockSpec((_TC3_BLK,), lambda i: (i,)),
        out_shape=jax.ShapeDtypeStruct((L,), _f32),
    )(a, b)


def kernel(x_user, x_job, edge_index_u2j, edge_index_j2u, edge_label_index,
           W1l_u2j, W1r_u2j, b1_u2j, W1l_j2u, W1r_j2u, b1_j2u,
           W2l_u2j, W2r_u2j, b2_u2j, W2l_j2u, W2r_j2u, b2_j2u):
    def pack_edges(v, fill):
        v2 = v.astype(_i32).reshape(NS, EPT)
        v2 = jnp.pad(v2, ((0, 0), (0, EPTB - EPT)), constant_values=fill)
        return v2.reshape(NS, ITERS, CH)

    su2j = pack_edges(edge_index_u2j[0], 0)
    du2j = pack_edges(edge_index_u2j[1], SINK)
    sj2u = pack_edges(edge_index_j2u[0], 0)
    dj2u = pack_edges(edge_index_j2u[1], SINK)
    la = edge_label_index[0].astype(_i32)
    lb = edge_label_index[1].astype(_i32)
    xu = jnp.pad(x_user, ((0, NP - N), (0, 0)))
    xj = jnp.pad(x_job, ((0, NP - N), (0, 0)))
    znd = jnp.zeros((NP, D), _f32)
    ones = jnp.ones((CH, D), _f32)
    b1j = b1_u2j.reshape(1, D)
    b1u = b1_j2u.reshape(1, D)
    b2j = b2_u2j.reshape(1, D)
    b2u = b2_j2u.reshape(1, D)

    summ_j, summ_u, cnt_j, cnt_u = _get_agg(True)(
        xu, xj, su2j, du2j, sj2u, dj2u, znd, ones)
    h_job, h_user = _tc_sage(
        summ_j, cnt_j, xj, W1l_u2j, b1j, W1r_u2j,
        summ_u, cnt_u, xu, W1l_j2u, b1u, W1r_j2u,
        relu=True, normalize=False)

    summ2_j, summ2_u = _get_agg(False)(
        h_user, h_job, su2j, du2j, sj2u, dj2u, znd)
    zn_job, zn_user = _tc_sage(
        summ2_j, cnt_j, h_job, W2l_u2j, b2j, W2r_u2j,
        summ2_u, cnt_u, h_user, W2l_j2u, b2u, W2r_j2u,
        relu=False, normalize=True)

    ga, gb = _get_gather()(zn_user, zn_job, la, lb)
    return _tc_rowdot(ga, gb)
